# baseline (speedup 1.0000x reference)
import numpy as np
import jax
import jax.numpy as jnp
from jax.experimental import pallas as pl
from jax.experimental.pallas import tpu as pltpu

B, T, S = 4096, 10000, 128
BR = 2048
NBLK = B // BR


def _subsample_taps():
    max_logspace = np.log10(T - 1)
    idx = np.round(np.logspace(0.0, max_logspace, S, endpoint=True), 1).astype(int)
    idx[0] = 0
    return idx.astype(np.int32)


def _build_plan():
    taps = _subsample_taps()
    tiles = sorted(set(int(t) // 128 for t in taps))
    tile_pos = {c: k for k, c in enumerate(tiles)}
    w = np.zeros((len(tiles), 128, S), np.float32)
    for j, t in enumerate(taps):
        t = int(t)
        w[tile_pos[t // 128], t % 128, j] = 1.0
    return np.asarray(tiles, np.int32), w


_TILES, _W = _build_plan()
NT = len(_TILES)


def _body(tiles_ref, x_ref, w_ref, o_ref):
    k = pl.program_id(1)

    @pl.when(k == 0)
    def _init():
        o_ref[...] = jnp.zeros_like(o_ref)

    valid = T - tiles_ref[k] * 128
    cols = jax.lax.broadcasted_iota(jnp.int32, (1, 128), 1)
    xv = jnp.where(cols < valid, x_ref[...], 0.0)
    o_ref[...] += jnp.dot(xv, w_ref[0], preferred_element_type=jnp.float32)


_gather = pl.pallas_call(
    _body,
    grid_spec=pltpu.PrefetchScalarGridSpec(
        num_scalar_prefetch=1,
        grid=(NBLK, NT),
        in_specs=[
            pl.BlockSpec((BR, 128), lambda i, k, tr: (i, tr[k])),
            pl.BlockSpec((1, 128, S), lambda i, k, tr: (k, 0, 0)),
        ],
        out_specs=pl.BlockSpec((BR, S), lambda i, k, tr: (i, 0)),
    ),
    out_shape=jax.ShapeDtypeStruct((B, S), jnp.float32),
    compiler_params=pltpu.CompilerParams(
        dimension_semantics=("parallel", "arbitrary"),
    ),
)


@jax.jit
def kernel(x):
    x2d = jnp.squeeze(x, axis=1)
    return _gather(jnp.asarray(_TILES), x2d, jnp.asarray(_W))


# speedup vs baseline: 1.2105x; 1.2105x over previous
import numpy as np
import jax
import jax.numpy as jnp
from jax.experimental import pallas as pl
from jax.experimental.pallas import tpu as pltpu

B, T, S = 4096, 10000, 128
BR = 1024
NBLK = B // BR
EDGE_TILE = T // 128
EDGE_VALID = T - EDGE_TILE * 128


def _subsample_taps():
    max_logspace = np.log10(T - 1)
    idx = np.round(np.logspace(0.0, max_logspace, S, endpoint=True), 1).astype(int)
    idx[0] = 0
    return idx.astype(np.int32)


def _build_plan():
    taps = _subsample_taps()
    tiles = sorted(set(int(t) // 128 for t in taps))
    in_tiles = [c for c in tiles if c != EDGE_TILE]
    runs = []
    for c in in_tiles:
        if runs and runs[-1][0] + runs[-1][1] == c:
            runs[-1] = (runs[-1][0], runs[-1][1] + 1)
        else:
            runs.append((c, 1))
    order = [c for (c0, n) in runs for c in range(c0, c0 + n)]
    pos = {c: k for k, c in enumerate(order)}
    nt = len(order)
    w = np.zeros((nt + 1, 128, S), np.float32)
    for j, t in enumerate(taps):
        t = int(t)
        c = t // 128
        k = nt if c == EDGE_TILE else pos[c]
        w[k, t % 128, j] = 1.0
    return [(c0, n) for (c0, n) in runs], w


_RUNS, _W = _build_plan()
NTIN = _W.shape[0] - 1
NRUNS = len(_RUNS)


def _body(x_hbm, xe_ref, w_ref, o_ref, *scratch):
    sems = scratch[-1]
    bufs = scratch[:-1]
    i = pl.program_id(0)
    slot = jax.lax.rem(i, 2)

    def start_runs(blk, slot_):
        r0 = blk * BR
        for r, (c0, n) in enumerate(_RUNS):
            pltpu.make_async_copy(
                x_hbm.at[pl.ds(r0, BR), pl.ds(c0 * 128, n * 128)],
                bufs[r].at[slot_],
                sems.at[r, slot_],
            ).start()

    def wait_runs(slot_):
        for r, (c0, n) in enumerate(_RUNS):
            pltpu.make_async_copy(
                x_hbm.at[pl.ds(0, BR), pl.ds(c0 * 128, n * 128)],
                bufs[r].at[slot_],
                sems.at[r, slot_],
            ).wait()

    @pl.when(i == 0)
    def _first():
        start_runs(0, 0)

    wait_runs(slot)

    @pl.when(i + 1 < NBLK)
    def _prefetch():
        start_runs(i + 1, 1 - slot)

    lanes = jax.lax.broadcasted_iota(jnp.int32, (1, 128), 1)
    xe = jnp.where(lanes < EDGE_VALID, xe_ref[...], 0.0)
    acc = jnp.dot(xe, w_ref[NTIN], preferred_element_type=jnp.float32)
    k = 0
    for r, (c0, n) in enumerate(_RUNS):
        buf = bufs[r]
        for t in range(n):
            acc += jnp.dot(
                buf[slot, :, t * 128:(t + 1) * 128],
                w_ref[k],
                preferred_element_type=jnp.float32,
            )
            k += 1
    o_ref[...] = acc


_gather = pl.pallas_call(
    _body,
    grid=(NBLK,),
    in_specs=[
        pl.BlockSpec(memory_space=pltpu.HBM),
        pl.BlockSpec((BR, 128), lambda i: (i, EDGE_TILE)),
        pl.BlockSpec((NTIN + 1, 128, S), lambda i: (0, 0, 0)),
    ],
    out_specs=pl.BlockSpec((BR, S), lambda i: (i, 0)),
    out_shape=jax.ShapeDtypeStruct((B, S), jnp.float32),
    scratch_shapes=(
        [pltpu.VMEM((2, BR, 128 * n), jnp.float32) for (_, n) in _RUNS]
        + [pltpu.SemaphoreType.DMA((NRUNS, 2))]
    ),
)


@jax.jit
def kernel(x):
    x2d = jnp.squeeze(x, axis=1)
    return _gather(x2d, x2d, jnp.asarray(_W))
